# baseline (speedup 1.0000x reference)
import functools

import jax
import jax.numpy as jnp
from jax import lax
from jax.experimental import pallas as pl
from jax.experimental.pallas import tpu as pltpu
from jax.experimental.pallas import tpu_sc as plsc

N_EMBD = 1024
VOCAB = 5
NPAIR = VOCAB * VOCAB
NPAD = 32

ROWS = 4 * 8192
NW = 32
RPW = ROWS // NW
CB = 32
NCH = RPW // CB
LANES = 16


def _combine_body(d_ref, p_ref, out_ref):
    d = d_ref[...]
    p = p_ref[...]
    comb = (d[:, None, :] + p[None, :, :]).reshape(NPAIR, N_EMBD)
    pad = jnp.zeros((NPAD - NPAIR, N_EMBD), jnp.float32)
    padded = jnp.concatenate([comb, pad], axis=0)
    out_ref[...] = jnp.broadcast_to(
        padded[None], (NW, NPAD, N_EMBD)
    ).reshape(NW * NPAD, N_EMBD)


def _combine(diaoemb_weight, phemb_weight):
    return pl.pallas_call(
        _combine_body,
        out_shape=jax.ShapeDtypeStruct((NW * NPAD, N_EMBD), jnp.float32),
    )(diaoemb_weight, phemb_weight)


def _sc_body(comb_hbm, x_hbm, diao_hbm, out_hbm, xv, dv, idxv, bufs, comb_v,
             wsem0, wsem1):
    sid = lax.axis_index("s")
    wid = sid * 2 + lax.axis_index("c")
    base = wid * RPW

    pltpu.sync_copy(comb_hbm.at[pl.ds(wid * NPAD, NPAD)], comb_v)

    pltpu.sync_copy(x_hbm.at[pl.ds(base, RPW)], xv)
    pltpu.sync_copy(diao_hbm.at[pl.ds(base, RPW)], dv)

    for k in range(RPW // LANES):
        s = pl.ds(k * LANES, LANES)
        idxv[s] = dv[s] * VOCAB + xv[s]

    wsems = (wsem0, wsem1)

    @pl.loop(0, NCH, step=2)
    def chunk_pair(i):
        for b in range(2):
            c = i + b

            @pl.when(i > 0)
            def _wait_prev():
                pltpu.make_async_copy(
                    bufs.at[b], out_hbm.at[pl.ds(base, CB)], wsems[b]
                ).wait()

            @plsc.parallel_loop(0, CB, unroll=4)
            def fill_row(r):
                iv = idxv[pl.ds(c * CB + r, LANES)]
                row = iv[0]
                for k in range(N_EMBD // LANES):
                    s = pl.ds(k * LANES, LANES)
                    bufs[b, r, s] = comb_v[row, s]

            pltpu.async_copy(
                bufs.at[b], out_hbm.at[pl.ds(base + c * CB, CB)], wsems[b]
            )

    for b in range(2):
        pltpu.make_async_copy(
            bufs.at[b], out_hbm.at[pl.ds(base, CB)], wsems[b]
        ).wait()


_sc_lookup = functools.partial(
    pl.kernel,
    out_type=jax.ShapeDtypeStruct((ROWS, N_EMBD), jnp.float32),
    mesh=plsc.VectorSubcoreMesh(core_axis_name="c", subcore_axis_name="s"),
    scratch_types=[
        pltpu.VMEM((RPW,), jnp.int32),
        pltpu.VMEM((RPW,), jnp.int32),
        pltpu.VMEM((RPW + LANES,), jnp.int32),
        pltpu.VMEM((2, CB, N_EMBD), jnp.float32),
        pltpu.VMEM((NPAD, N_EMBD), jnp.float32),
        pltpu.SemaphoreType.DMA,
        pltpu.SemaphoreType.DMA,
    ],
)(_sc_body)


@jax.jit
def kernel(x, diao, diaoemb_weight, phemb_weight):
    comb = _combine(diaoemb_weight, phemb_weight)
    xf = x.reshape(ROWS).astype(jnp.int32)
    df = diao.reshape(ROWS).astype(jnp.int32)
    out = _sc_lookup(comb, xf, df)
    return out.reshape(x.shape[0], x.shape[1], N_EMBD)


# speedup vs baseline: 1.4838x; 1.4838x over previous
import functools

import jax
import jax.numpy as jnp
from jax import lax
from jax.experimental import pallas as pl
from jax.experimental.pallas import tpu as pltpu
from jax.experimental.pallas import tpu_sc as plsc

N_EMBD = 1024
VOCAB = 5
NPAIR = VOCAB * VOCAB
NPAD = 32

ROWS = 4 * 8192
NW = 32
RPW = ROWS // NW
CB = 32
NCH = RPW // CB
LANES = 16


def _combine_body(d_ref, p_ref, out_ref):
    d = d_ref[...]
    p = p_ref[...]
    comb = (d[:, None, :] + p[None, :, :]).reshape(NPAIR, N_EMBD)
    pad = jnp.zeros((NPAD - NPAIR, N_EMBD), jnp.float32)
    padded = jnp.concatenate([comb, pad], axis=0)
    out_ref[...] = jnp.broadcast_to(
        padded[None], (NW, NPAD, N_EMBD)
    ).reshape(NW * NPAD, N_EMBD)


def _combine(diaoemb_weight, phemb_weight):
    return pl.pallas_call(
        _combine_body,
        out_shape=jax.ShapeDtypeStruct((NW * NPAD, N_EMBD), jnp.float32),
    )(diaoemb_weight, phemb_weight)


def _sc_body(comb_hbm, x_hbm, diao_hbm, out_hbm, xv, dv, idxv, bufs, comb_v,
             wsem0, wsem1):
    sid = lax.axis_index("s")
    wid = sid * 2 + lax.axis_index("c")
    base = wid * RPW

    pltpu.sync_copy(comb_hbm.at[pl.ds(wid * NPAD, NPAD)], comb_v)

    pltpu.sync_copy(x_hbm.at[pl.ds(base, RPW)], xv)
    pltpu.sync_copy(diao_hbm.at[pl.ds(base, RPW)], dv)

    for k in range(RPW // LANES):
        s = pl.ds(k * LANES, LANES)
        idxv[s] = dv[s] * VOCAB + xv[s]

    wsems = (wsem0, wsem1)

    @pl.loop(0, NCH, step=2)
    def chunk_pair(i):
        for b in range(2):
            c = i + b

            @pl.when(i > 0)
            def _wait_prev():
                pltpu.make_async_copy(
                    bufs.at[b], out_hbm.at[pl.ds(base, CB)], wsems[b]
                ).wait()

            @plsc.parallel_loop(0, CB, unroll=1)
            def fill_row(r):
                iv = idxv[pl.ds(c * CB + r, LANES)]
                row = iv[0]
                for k in range(N_EMBD // LANES):
                    s = pl.ds(k * LANES, LANES)
                    bufs[b, r, s] = comb_v[row, s]

            pltpu.async_copy(
                bufs.at[b], out_hbm.at[pl.ds(base + c * CB, CB)], wsems[b]
            )

    for b in range(2):
        pltpu.make_async_copy(
            bufs.at[b], out_hbm.at[pl.ds(base, CB)], wsems[b]
        ).wait()


_sc_lookup = functools.partial(
    pl.kernel,
    out_type=jax.ShapeDtypeStruct((ROWS, N_EMBD), jnp.float32),
    mesh=plsc.VectorSubcoreMesh(core_axis_name="c", subcore_axis_name="s"),
    scratch_types=[
        pltpu.VMEM((RPW,), jnp.int32),
        pltpu.VMEM((RPW,), jnp.int32),
        pltpu.VMEM((RPW + LANES,), jnp.int32),
        pltpu.VMEM((2, CB, N_EMBD), jnp.float32),
        pltpu.VMEM((NPAD, N_EMBD), jnp.float32),
        pltpu.SemaphoreType.DMA,
        pltpu.SemaphoreType.DMA,
    ],
)(_sc_body)


@jax.jit
def kernel(x, diao, diaoemb_weight, phemb_weight):
    comb = _combine(diaoemb_weight, phemb_weight)
    xf = x.reshape(ROWS).astype(jnp.int32)
    df = diao.reshape(ROWS).astype(jnp.int32)
    out = _sc_lookup(comb, xf, df)
    return out.reshape(x.shape[0], x.shape[1], N_EMBD)
